# baseline (device time: 15594 ns/iter reference)
import jax
import jax.numpy as jnp
from jax import lax
from jax.experimental import pallas as pl
from jax.experimental.pallas import tpu as pltpu

N_DEV = 8
NR = 2


def kernel(Q, K, V):
    b, sq, h, d = Q.shape
    skv = K.shape[1]
    scale = d ** -0.5
    bh = b * h
    bhd = bh * d
    rbh = bh // NR
    rrows = rbh * d

    Qf = Q.reshape(bh, d)
    Kt = jnp.transpose(K, (0, 2, 3, 1)).reshape(bhd, skv).astype(jnp.bfloat16)
    Vt = jnp.transpose(V, (0, 2, 3, 1)).reshape(bhd, skv)

    def body(q_ref, k_ref, v_ref, out_ref, mine_ref, comm_ref, send_sems, recv_sems):
        my = lax.axis_index("i")

        bsem = pltpu.get_barrier_semaphore()
        for off in range(1, N_DEV):
            pl.semaphore_signal(
                bsem, inc=1,
                device_id=((my + off) % N_DEV,),
                device_id_type=pl.DeviceIdType.MESH,
            )

        q2 = q_ref[:] * scale
        qt = jnp.broadcast_to(q2[:, None, :], (bh, bh, d)).reshape(bh, bhd)
        mbd = (
            lax.broadcasted_iota(jnp.int32, (bh, bhd), 1) // d
            == lax.broadcasted_iota(jnp.int32, (bh, bhd), 0)
        )
        qbd = jnp.where(mbd, qt, 0.0).astype(jnp.bfloat16)

        rdmas = []
        for r in range(NR):
            s = lax.dot_general(
                qbd[r * rbh:(r + 1) * rbh], k_ref[:],
                (((1,), (0,)), ((), ())),
                preferred_element_type=jnp.float32,
            )
            pc = jnp.exp(s)
            den = jnp.sum(pc, axis=1, keepdims=True)
            p3 = jnp.broadcast_to(pc[:, None, :], (rbh, d, skv)).reshape(rrows, skv)
            vc = v_ref[r * rrows:(r + 1) * rrows]
            o2 = jnp.sum(vc * p3, axis=1).reshape(rbh, d)
            packed = jnp.concatenate(
                [o2, jnp.broadcast_to(den, (rbh, d))],
                axis=1,
            ).astype(jnp.bfloat16)
            mine_ref[r] = packed
            if r == 0:
                pl.semaphore_wait(bsem, N_DEV - 1)
            for off in range(1, N_DEV):
                rdma = pltpu.make_async_remote_copy(
                    src_ref=mine_ref.at[r],
                    dst_ref=comm_ref.at[r, off - 1],
                    send_sem=send_sems.at[r * (N_DEV - 1) + off - 1],
                    recv_sem=recv_sems.at[r * (N_DEV - 1) + off - 1],
                    device_id=((my + off) % N_DEV,),
                    device_id_type=pl.DeviceIdType.MESH,
                )
                rdma.start()
                rdmas.append(rdma)

        for rdma in rdmas:
            rdma.wait_recv()

        outs = []
        for r in range(NR):
            acc = mine_ref[r].astype(jnp.float32)
            for slot in range(N_DEV - 1):
                acc = acc + comm_ref[r, slot].astype(jnp.float32)
            outs.append(acc[:, :d] / acc[:, d:])
        out_ref[:] = jnp.concatenate(outs, axis=0).reshape(b, sq, h, d)

        for rdma in rdmas:
            rdma.wait_send()

    return pl.pallas_call(
        body,
        out_shape=jax.ShapeDtypeStruct((b, sq, h, d), jnp.float32),
        in_specs=[pl.BlockSpec(memory_space=pltpu.VMEM)] * 3,
        out_specs=pl.BlockSpec(memory_space=pltpu.VMEM),
        scratch_shapes=[
            pltpu.VMEM((NR, rbh, 2 * d), jnp.bfloat16),
            pltpu.VMEM((NR, N_DEV - 1, rbh, 2 * d), jnp.bfloat16),
            pltpu.SemaphoreType.DMA((NR * (N_DEV - 1),)),
            pltpu.SemaphoreType.DMA((NR * (N_DEV - 1),)),
        ],
        compiler_params=pltpu.CompilerParams(collective_id=0),
    )(Qf, Kt, Vt)


# device time: 14749 ns/iter; 1.0573x vs baseline; 1.0573x over previous
import jax
import jax.numpy as jnp
from jax import lax
from jax.experimental import pallas as pl
from jax.experimental.pallas import tpu as pltpu

N_DEV = 8
NR = 2


def kernel(Q, K, V):
    b, sq, h, d = Q.shape
    skv = K.shape[1]
    scale = d ** -0.5
    bh = b * h
    bhd = bh * d
    rbh = bh // NR
    rrows = rbh * d

    Qf = Q.reshape(bh, d)
    Kt = jnp.transpose(K, (0, 2, 3, 1)).reshape(bhd, skv).astype(jnp.bfloat16)
    Vt = jnp.transpose(V, (0, 2, 3, 1)).reshape(bhd, skv)

    def body(q_ref, k_ref, v_ref, out_ref, mine_ref, comm_ref, send_sems, recv_sems):
        my = lax.axis_index("i")

        bsem = pltpu.get_barrier_semaphore()
        for off in range(1, N_DEV):
            pl.semaphore_signal(
                bsem, inc=1,
                device_id=((my + off) % N_DEV,),
                device_id_type=pl.DeviceIdType.MESH,
            )

        q2 = q_ref[:] * scale
        qt = jnp.broadcast_to(q2[:, None, :], (bh, bh, d)).reshape(bh, bhd)
        mbd = (
            lax.broadcasted_iota(jnp.int32, (bh, bhd), 1) // d
            == lax.broadcasted_iota(jnp.int32, (bh, bhd), 0)
        )
        qbd = jnp.where(mbd, qt, 0.0).astype(jnp.bfloat16)

        s = lax.dot_general(
            qbd, k_ref[:],
            (((1,), (0,)), ((), ())),
            preferred_element_type=jnp.float32,
        )
        p = jnp.exp(s)
        den = jnp.sum(p, axis=1, keepdims=True)

        rdmas = []
        for r in range(NR):
            pc = p[r * rbh:(r + 1) * rbh]
            p3 = jnp.broadcast_to(pc[:, None, :], (rbh, d, skv)).reshape(rrows, skv)
            vc = v_ref[r * rrows:(r + 1) * rrows]
            o2 = jnp.sum(vc * p3, axis=1).reshape(rbh, d)
            packed = jnp.concatenate(
                [o2, jnp.broadcast_to(den[r * rbh:(r + 1) * rbh], (rbh, d))],
                axis=1,
            ).astype(jnp.bfloat16)
            mine_ref[r] = packed
            if r == 0:
                pl.semaphore_wait(bsem, N_DEV - 1)
            for off in range(1, N_DEV):
                rdma = pltpu.make_async_remote_copy(
                    src_ref=mine_ref.at[r],
                    dst_ref=comm_ref.at[r, off - 1],
                    send_sem=send_sems.at[r * (N_DEV - 1) + off - 1],
                    recv_sem=recv_sems.at[r * (N_DEV - 1) + off - 1],
                    device_id=((my + off) % N_DEV,),
                    device_id_type=pl.DeviceIdType.MESH,
                )
                rdma.start()
                rdmas.append(rdma)

        for rdma in rdmas:
            rdma.wait_recv()

        outs = []
        for r in range(NR):
            acc = mine_ref[r].astype(jnp.float32)
            for slot in range(N_DEV - 1):
                acc = acc + comm_ref[r, slot].astype(jnp.float32)
            outs.append(acc[:, :d] / acc[:, d:])
        out_ref[:] = jnp.concatenate(outs, axis=0).reshape(b, sq, h, d)

        for rdma in rdmas:
            rdma.wait_send()

    return pl.pallas_call(
        body,
        out_shape=jax.ShapeDtypeStruct((b, sq, h, d), jnp.float32),
        in_specs=[pl.BlockSpec(memory_space=pltpu.VMEM)] * 3,
        out_specs=pl.BlockSpec(memory_space=pltpu.VMEM),
        scratch_shapes=[
            pltpu.VMEM((NR, rbh, 2 * d), jnp.bfloat16),
            pltpu.VMEM((NR, N_DEV - 1, rbh, 2 * d), jnp.bfloat16),
            pltpu.SemaphoreType.DMA((NR * (N_DEV - 1),)),
            pltpu.SemaphoreType.DMA((NR * (N_DEV - 1),)),
        ],
        compiler_params=pltpu.CompilerParams(collective_id=0),
    )(Qf, Kt, Vt)
